# baseline (speedup 1.0000x reference)
import jax
import jax.numpy as jnp
from jax.experimental import pallas as pl
from jax.experimental.pallas import tpu as pltpu


def _make_body(nb, k_chunks):
    def body(x_ref, o_ref, vmax_ref, varg_ref, iota_ref):
        big = jnp.int32(1 << 30)
        i = pl.program_id(0)
        j = pl.program_id(1)
        k = pl.program_id(2)

        @pl.when((i == 0) & (j == 0) & (k == 0))
        def _init_iota():
            iota_ref[...] = jax.lax.broadcasted_iota(jnp.int32, (nb, 16), 0)

        @pl.when(j == 0)
        def _reduce():
            @pl.when(k == 0)
            def _init():
                vmax_ref[...] = jnp.full((1, 16), -jnp.inf, jnp.float32)
                varg_ref[...] = jnp.zeros((1, 16), jnp.int32)

            data = x_ref[0]
            cmx = jnp.max(data, axis=0, keepdims=True)
            lam = jnp.min(jnp.where(data == cmx, iota_ref[...], big),
                          axis=0, keepdims=True)
            better = cmx > vmax_ref[...]
            vmax_ref[...] = jnp.where(better, cmx, vmax_ref[...])
            varg_ref[...] = jnp.where(better, lam + k * nb, varg_ref[...])

        @pl.when(j == 1)
        def _write():
            tgt = varg_ref[...] - k * nb
            o_ref[0] = (iota_ref[...] == tgt).astype(jnp.float32)

    return body


def kernel(x):
    b, n, m = x.shape
    k_chunks = 8
    nb = n // k_chunks

    out = pl.pallas_call(
        _make_body(nb, k_chunks),
        grid=(b, 2, k_chunks),
        in_specs=[pl.BlockSpec(
            (1, nb, m),
            lambda i, j, k: (i, jnp.where(j == 0, k, 0), 0))],
        out_specs=pl.BlockSpec(
            (1, nb, m),
            lambda i, j, k: (i, jnp.where(j == 0, 0, k), 0)),
        out_shape=jax.ShapeDtypeStruct((b, n, m), jnp.float32),
        scratch_shapes=[
            pltpu.VMEM((1, 16), jnp.float32),
            pltpu.VMEM((1, 16), jnp.int32),
            pltpu.VMEM((nb, 16), jnp.int32),
        ],
    )(x)
    return out


# speedup vs baseline: 1.1704x; 1.1704x over previous
import dataclasses
import functools

import jax
import jax.numpy as jnp
from jax import lax
from jax.experimental import pallas as pl
from jax.experimental.pallas import tpu as pltpu
from jax.experimental.pallas import tpu_sc as plsc

_CH = 1024


def _sc_hardmax(n, m, n_workers):
    nch = n // _CH
    mesh = plsc.VectorSubcoreMesh(core_axis_name="c", subcore_axis_name="s",
                                  num_cores=2, num_subcores=16)
    cp = pltpu.CompilerParams()
    if "needs_layout_passes" in pltpu.CompilerParams.__dataclass_fields__:
        cp = dataclasses.replace(cp, needs_layout_passes=False)
    if "use_tc_tiling_on_sc" in pltpu.CompilerParams.__dataclass_fields__:
        cp = dataclasses.replace(cp, use_tc_tiling_on_sc=False)

    @functools.partial(
        pl.kernel,
        compiler_params=cp,
        out_type=jax.ShapeDtypeStruct((n_workers * n, m), jnp.float32),
        mesh=mesh,
        scratch_types=[
            pltpu.VMEM((_CH, 16), jnp.float32),
            pltpu.VMEM((_CH, 16), jnp.float32),
            pltpu.VMEM((_CH, 16), jnp.float32),
            pltpu.VMEM((16, 16), jnp.float32),
            pltpu.VMEM((16,), jnp.float32),
            pltpu.VMEM((16,), jnp.int32),
            pltpu.SemaphoreType.DMA,
            pltpu.SemaphoreType.DMA,
            pltpu.SemaphoreType.DMA,
        ],
    )
    def k(x_hbm, o_hbm, buf0, buf1, zbuf, pay, gmax, gidx,
          rsem, zsem, ssem):
        big = jnp.int32(1 << 30)
        wid = lax.axis_index("s") * 2 + lax.axis_index("c")
        base = wid * n

        zrow = jnp.zeros((16,), jnp.float32)

        @pl.loop(0, _CH)
        def _(r):
            zbuf[r] = zrow

        gmax[...] = jnp.full((16,), -jnp.inf, jnp.float32)
        gidx[...] = jnp.zeros((16,), jnp.int32)

        def process(buf, c):
            chunk_base = c * _CH

            neg = jnp.full((16,), -jnp.inf, jnp.float32)

            def body_a(t, accs):
                r = t * 8
                return tuple(
                    jnp.maximum(accs[i], buf[r + i]) for i in range(8))

            accs = lax.fori_loop(0, _CH // 8, body_a, (neg,) * 8)
            cmx = accs[0]
            for i in range(1, 8):
                cmx = jnp.maximum(cmx, accs[i])

            gm = gmax[...]
            better = cmx > gm

            @pl.when(jnp.any(better))
            def _():
                def body_b(t, bidx):
                    eqm = buf[t] == cmx
                    rfull = jnp.full((16,), t, jnp.int32)
                    return jnp.minimum(bidx, jnp.where(eqm, rfull, big))

                bidx = lax.fori_loop(0, _CH, body_b,
                                     jnp.full((16,), big, jnp.int32))
                gmax[...] = jnp.where(better, cmx, gm)
                gidx[...] = jnp.where(better, bidx + chunk_base, gidx[...])

        h0 = pltpu.async_copy(x_hbm.at[pl.ds(base, _CH)], buf0, rsem)
        h1 = pltpu.async_copy(x_hbm.at[pl.ds(base + _CH, _CH)], buf1, rsem)

        @pl.loop(0, nch // 2)
        def _(g):
            c0 = g * 2
            pltpu.async_copy(
                zbuf, o_hbm.at[pl.ds(base + c0 * _CH, _CH)], zsem)
            pltpu.async_copy(
                zbuf, o_hbm.at[pl.ds(base + (c0 + 1) * _CH, _CH)], zsem)
            pltpu.make_async_copy(
                x_hbm.at[pl.ds(base, _CH)], buf0, rsem).wait()
            process(buf0, c0)

            @pl.when(g < nch // 2 - 1)
            def _():
                pltpu.async_copy(
                    x_hbm.at[pl.ds(base + (c0 + 2) * _CH, _CH)], buf0, rsem)

            pltpu.make_async_copy(
                x_hbm.at[pl.ds(base, _CH)], buf1, rsem).wait()
            process(buf1, c0 + 1)

            @pl.when(g < nch // 2 - 1)
            def _():
                pltpu.async_copy(
                    x_hbm.at[pl.ds(base + (c0 + 3) * _CH, _CH)], buf1, rsem)

        gvec = gidx[...]
        gdst = gvec + base
        dn = lax.GatherDimensionNumbers(
            offset_dims=(), collapsed_slice_dims=(0,), start_index_map=(0,))
        for j in range(16):
            bc = lax.gather(
                gvec, jnp.full((16, 1), j, jnp.int32), dn, (1,),
                mode=lax.GatherScatterMode.PROMISE_IN_BOUNDS)
            pay[j] = jnp.where(gvec == bc, 1.0, 0.0).astype(jnp.float32)

        @pl.loop(0, nch)
        def _(c):
            pltpu.make_async_copy(
                zbuf, o_hbm.at[pl.ds(base, _CH)], zsem).wait()

        pltpu.async_copy(pay, o_hbm.at[gdst], ssem).wait()

    return k


def kernel(x):
    b, n, m = x.shape
    xf = x.reshape(b * n, m)
    out = _sc_hardmax(n, m, b)(xf)
    return out.reshape(b, n, m)
